# Initial kernel scaffold; baseline (speedup 1.0000x reference)
import jax
import jax.numpy as jnp
from jax.experimental import pallas as pl


def kernel(input, target):
    raise NotImplementedError("write your pallas kernel here")


# speedup vs baseline: 7.6091x; 7.6091x over previous
import functools

import jax
import jax.numpy as jnp
from jax import lax
from jax.experimental import pallas as pl
from jax.experimental.pallas import tpu as pltpu
from jax.experimental.pallas import tpu_sc as plsc

B_BITS = 17
NB = 1 << B_BITS
NIMG = 8
N = 512 * 512
NCORE = 2
NSUB = 16
IMGS_PER_CORE = NIMG // NCORE
CHUNK = N // NSUB
ROWS = CHUNK // 128
SLICE = NB // NSUB
SUBB = SLICE // 4


def _lanes():
    return lax.iota(jnp.int32, 16)


def _splat(x):
    return jnp.broadcast_to(x, (16,))


def _body(logits_hbm, labels_hbm, out_hbm, ebuf, tbuf, idxbuf, onesbuf,
          hp, ht, wp, wn, wbuf, sbuf, sumsbuf,
          hist_pos, hist_tot, wtab, sums_sh):
    c = lax.axis_index("c")
    s = lax.axis_index("s")
    lanes = _lanes()
    zeros16 = jnp.zeros((16,), jnp.float32)
    ones16 = jnp.ones((16,), jnp.float32)

    for u in range(8):
        onesbuf[pl.ds(u * 16, 16)] = ones16

    def image_step(ii, acc):
        img = c * IMGS_PER_CORE + ii

        def zero_wbuf(i, carry):
            wbuf[pl.ds(i * 16, 16)] = zeros16
            return carry
        lax.fori_loop(0, CHUNK // 16, zero_wbuf, 0)

        pltpu.sync_copy(logits_hbm.at[img, s], ebuf)
        pltpu.sync_copy(labels_hbm.at[img, s], tbuf)
        pltpu.sync_copy(wbuf.at[pl.ds(0, SLICE)],
                        hist_pos.at[pl.ds(s * SLICE, SLICE)])
        pltpu.sync_copy(wbuf.at[pl.ds(0, SLICE)],
                        hist_tot.at[pl.ds(s * SLICE, SLICE)])
        plsc.subcore_barrier()

        def arow(j, carry):
            for u in range(8):
                sl = pl.ds(u * 16, 16)
                fl = pl.ds(j * 128 + u * 16, 16)
                x = ebuf[fl]
                t = tbuf[j, sl]
                e = 1.0 - x * (2.0 * t - 1.0)
                ebuf[fl] = e
                bu = plsc.bitcast(e, jnp.uint32)
                k = jnp.where(bu >= jnp.uint32(0x80000000),
                              ~bu, bu | jnp.uint32(0x80000000))
                h = (k >> jnp.uint32(32 - B_BITS)).astype(jnp.int32)
                idxbuf[j, sl] = h
            pltpu.sync_copy(tbuf.at[j], hist_pos.at[idxbuf.at[j]], add=True)
            pltpu.sync_copy(onesbuf, hist_tot.at[idxbuf.at[j]], add=True)
            return carry
        lax.fori_loop(0, ROWS, arow, 0)
        plsc.subcore_barrier()

        psum, tsum = zeros16, zeros16
        for sub in range(4):
            base = s * SLICE + sub * SUBB
            pltpu.sync_copy(hist_pos.at[pl.ds(base, SUBB)], hp)
            pltpu.sync_copy(hist_tot.at[pl.ds(base, SUBB)], ht)

            def srow(i, carry):
                ps, ts = carry
                return (ps + hp[pl.ds(i * 16, 16)], ts + ht[pl.ds(i * 16, 16)])
            psum, tsum = lax.fori_loop(0, SUBB // 16, srow, (psum, tsum))
        sp = jnp.sum(psum)
        st = jnp.sum(tsum)
        sbuf[...] = jnp.where(lanes == 0, _splat(sp),
                              jnp.where(lanes == 1, _splat(st), zeros16))
        pltpu.sync_copy(sbuf, sums_sh.at[s])
        plsc.subcore_barrier()
        pltpu.sync_copy(sums_sh, sumsbuf)

        Pv, cpv, ctv = zeros16, zeros16, zeros16
        for j2 in range(NSUB):
            row = sumsbuf[j2, pl.ds(0, 16)]
            spj = _splat(jnp.sum(jnp.where(lanes == 0, row, 0.0)))
            stj = _splat(jnp.sum(jnp.where(lanes == 1, row, 0.0)))
            Pv = Pv + spj
            gt = jnp.int32(j2) > s
            cpv = cpv + jnp.where(gt, spj, zeros16)
            ctv = ctv + jnp.where(gt, stj, zeros16)

        for sub in (3, 2, 1, 0):
            base = s * SLICE + sub * SUBB
            pltpu.sync_copy(hist_pos.at[pl.ds(base, SUBB)], hp)
            pltpu.sync_copy(hist_tot.at[pl.ds(base, SUBB)], ht)

            def wrow(i, carry):
                cp, ct = carry
                v = SUBB // 16 - 1 - i
                sl = pl.ds(v * 16, 16)
                hpv = hp[sl]
                htv = ht[sl]
                sfp = jnp.flip(jnp.cumsum(jnp.flip(hpv, 0)), 0) + cp
                sft = jnp.flip(jnp.cumsum(jnp.flip(htv, 0)), 0) + ct
                a = Pv + ((sft - htv) - (sfp - hpv))
                b = a + (htv - hpv)
                cc = Pv - sfp
                wn[sl] = jnp.where(a == 0.0,
                                   1.0 / jnp.maximum(b - a, 1.0),
                                   cc / jnp.maximum(a * b, 1.0))
                wp[sl] = 1.0 / jnp.maximum(a, 1.0)
                return (cp + _splat(jnp.sum(hpv)), ct + _splat(jnp.sum(htv)))
            cpv, ctv = lax.fori_loop(0, SUBB // 16, wrow, (cpv, ctv))
            pltpu.sync_copy(wn, wtab.at[pl.ds(base, SUBB)])
            pltpu.sync_copy(wp, wtab.at[pl.ds(NB + base, SUBB)])
        plsc.subcore_barrier()

        def crow(j, acc_in):
            for u in range(8):
                sl = pl.ds(u * 16, 16)
                t = tbuf[j, sl]
                idxbuf[j, sl] = idxbuf[j, sl] + jnp.int32(NB) * t.astype(jnp.int32)
            pltpu.sync_copy(wtab.at[idxbuf.at[j]], wbuf.at[pl.ds(j * 128, 128)])
            for u in range(8):
                fl = pl.ds(j * 128 + u * 16, 16)
                acc_in = acc_in + jnp.maximum(ebuf[fl], 0.0) * wbuf[fl]
            return acc_in
        return lax.fori_loop(0, ROWS, crow, acc)

    acc = lax.fori_loop(0, IMGS_PER_CORE, image_step, jnp.zeros((16,), jnp.float32))

    wid = c * NSUB + s
    sbuf[...] = acc
    pltpu.sync_copy(sbuf, out_hbm.at[pl.ds(wid * 16, 16)])


_sc_call = functools.partial(
    pl.kernel,
    out_type=jax.ShapeDtypeStruct((NCORE * NSUB * 16,), jnp.float32),
    mesh=plsc.VectorSubcoreMesh(core_axis_name="c", subcore_axis_name="s"),
    compiler_params=pltpu.CompilerParams(needs_layout_passes=False),
    scratch_types=[
        pltpu.VMEM((CHUNK,), jnp.float32),
        pltpu.VMEM((ROWS, 128), jnp.float32),
        pltpu.VMEM((ROWS, 128), jnp.int32),
        pltpu.VMEM((128,), jnp.float32),
        pltpu.VMEM((SUBB,), jnp.float32),
        pltpu.VMEM((SUBB,), jnp.float32),
        pltpu.VMEM((SUBB,), jnp.float32),
        pltpu.VMEM((SUBB,), jnp.float32),
        pltpu.VMEM((CHUNK,), jnp.float32),
        pltpu.VMEM((16,), jnp.float32),
        pltpu.VMEM((NSUB, 16), jnp.float32),
        pltpu.VMEM_SHARED((NB,), jnp.float32),
        pltpu.VMEM_SHARED((NB,), jnp.float32),
        pltpu.VMEM_SHARED((2 * NB,), jnp.float32),
        pltpu.VMEM_SHARED((NSUB, 16), jnp.float32),
    ],
)(_body)


def kernel(input, target):
    logits = input.reshape(NIMG, NSUB, CHUNK)
    labels = target.astype(jnp.float32).reshape(NIMG, NSUB, ROWS, 128)
    partials = _sc_call(logits, labels)
    return jnp.sum(partials) / NIMG


# speedup vs baseline: 12.5962x; 1.6554x over previous
import functools

import jax
import jax.numpy as jnp
from jax import lax
from jax.experimental import pallas as pl
from jax.experimental.pallas import tpu as pltpu
from jax.experimental.pallas import tpu_sc as plsc

B_BITS = 17
NB = 1 << B_BITS
NIMG = 8
N = 512 * 512
NCORE = 2
NSUB = 16
IMGS_PER_CORE = NIMG // NCORE
CHUNK = N // NSUB
ROWS = CHUNK // 128
SLICE = NB // NSUB
SUBB = SLICE // 4


def _lanes():
    return lax.iota(jnp.int32, 16)


def _splat(x):
    return jnp.broadcast_to(x, (16,))


def _body(logits_hbm, labels_hbm, out_hbm, ebuf, tbuf, idxbuf, onesbuf,
          hp, ht, wp, wn, wbuf, zbuf, sbuf, sumsbuf,
          hist_pos, hist_tot, wtab, sums_sh):
    c = lax.axis_index("c")
    s = lax.axis_index("s")
    lanes = _lanes()
    zeros16 = jnp.zeros((16,), jnp.float32)
    ones16 = jnp.ones((16,), jnp.float32)

    def fill_ones(i, carry):
        onesbuf[pl.ds(i * 16, 16)] = ones16
        return carry
    lax.fori_loop(0, CHUNK // 16, fill_ones, 0)

    def fill_zeros(i, carry):
        zbuf[pl.ds(i * 16, 16)] = zeros16
        return carry
    lax.fori_loop(0, SLICE // 32, fill_zeros, 0)

    def image_step(ii, acc):
        img = c * IMGS_PER_CORE + ii

        pltpu.sync_copy(logits_hbm.at[img, s], ebuf)
        pltpu.sync_copy(labels_hbm.at[img, s], tbuf)
        for half in range(2):
            off = s * SLICE + half * (SLICE // 2)
            pltpu.sync_copy(zbuf, hist_pos.at[pl.ds(off, SLICE // 2)])
            pltpu.sync_copy(zbuf, hist_tot.at[pl.ds(off, SLICE // 2)])
        plsc.subcore_barrier()

        def arow(j, carry):
            for u in range(8):
                sl = pl.ds(u * 16, 16)
                fl = pl.ds(j * 128 + u * 16, 16)
                x = ebuf[fl]
                t = tbuf[fl]
                e = 1.0 - x * (2.0 * t - 1.0)
                ebuf[fl] = e
                bu = plsc.bitcast(e, jnp.uint32)
                k = jnp.where(bu >= jnp.uint32(0x80000000),
                              ~bu, bu | jnp.uint32(0x80000000))
                h = (k >> jnp.uint32(32 - B_BITS)).astype(jnp.int32)
                idxbuf[fl] = h
            return carry
        lax.fori_loop(0, ROWS, arow, 0)
        pltpu.sync_copy(tbuf, hist_pos.at[idxbuf], add=True)
        pltpu.sync_copy(onesbuf, hist_tot.at[idxbuf], add=True)
        plsc.subcore_barrier()

        psum, tsum = zeros16, zeros16
        for sub in range(4):
            base = s * SLICE + sub * SUBB
            pltpu.sync_copy(hist_pos.at[pl.ds(base, SUBB)], hp)
            pltpu.sync_copy(hist_tot.at[pl.ds(base, SUBB)], ht)

            def srow(i, carry):
                ps, ts = carry
                return (ps + hp[pl.ds(i * 16, 16)], ts + ht[pl.ds(i * 16, 16)])
            psum, tsum = lax.fori_loop(0, SUBB // 16, srow, (psum, tsum))
        sp = jnp.sum(psum)
        st = jnp.sum(tsum)
        sbuf[...] = jnp.where(lanes == 0, _splat(sp),
                              jnp.where(lanes == 1, _splat(st), zeros16))
        pltpu.sync_copy(sbuf, sums_sh.at[s])
        plsc.subcore_barrier()
        pltpu.sync_copy(sums_sh, sumsbuf)

        Pv, cpv, ctv = zeros16, zeros16, zeros16
        for j2 in range(NSUB):
            row = sumsbuf[j2, pl.ds(0, 16)]
            spj = _splat(jnp.sum(jnp.where(lanes == 0, row, 0.0)))
            stj = _splat(jnp.sum(jnp.where(lanes == 1, row, 0.0)))
            Pv = Pv + spj
            gt = jnp.int32(j2) > s
            cpv = cpv + jnp.where(gt, spj, zeros16)
            ctv = ctv + jnp.where(gt, stj, zeros16)

        for sub in (3, 2, 1, 0):
            base = s * SLICE + sub * SUBB
            pltpu.sync_copy(hist_pos.at[pl.ds(base, SUBB)], hp)
            pltpu.sync_copy(hist_tot.at[pl.ds(base, SUBB)], ht)

            def wrow(i, carry):
                cp, ct = carry
                v = SUBB // 16 - 1 - i
                sl = pl.ds(v * 16, 16)
                hpv = hp[sl]
                htv = ht[sl]
                sfp = jnp.flip(jnp.cumsum(jnp.flip(hpv, 0)), 0) + cp
                sft = jnp.flip(jnp.cumsum(jnp.flip(htv, 0)), 0) + ct
                a = Pv + ((sft - htv) - (sfp - hpv))
                b = a + (htv - hpv)
                cc = Pv - sfp
                wn[sl] = jnp.where(a == 0.0,
                                   1.0 / jnp.maximum(b - a, 1.0),
                                   cc / jnp.maximum(a * b, 1.0))
                wp[sl] = 1.0 / jnp.maximum(a, 1.0)
                return (cp + _splat(jnp.sum(hpv)), ct + _splat(jnp.sum(htv)))
            cpv, ctv = lax.fori_loop(0, SUBB // 16, wrow, (cpv, ctv))
            pltpu.sync_copy(wn, wtab.at[pl.ds(base, SUBB)])
            pltpu.sync_copy(wp, wtab.at[pl.ds(NB + base, SUBB)])
        plsc.subcore_barrier()

        def cidx(j, carry):
            for u in range(8):
                fl = pl.ds(j * 128 + u * 16, 16)
                t = tbuf[fl]
                idxbuf[fl] = idxbuf[fl] + jnp.int32(NB) * t.astype(jnp.int32)
            return carry
        lax.fori_loop(0, ROWS, cidx, 0)
        pltpu.sync_copy(wtab.at[idxbuf], wbuf)

        def crow(j, acc_in):
            for u in range(8):
                fl = pl.ds(j * 128 + u * 16, 16)
                acc_in = acc_in + jnp.maximum(ebuf[fl], 0.0) * wbuf[fl]
            return acc_in
        return lax.fori_loop(0, ROWS, crow, acc)

    acc = lax.fori_loop(0, IMGS_PER_CORE, image_step, jnp.zeros((16,), jnp.float32))

    wid = c * NSUB + s
    sbuf[...] = acc
    pltpu.sync_copy(sbuf, out_hbm.at[pl.ds(wid * 16, 16)])


_sc_call = functools.partial(
    pl.kernel,
    out_type=jax.ShapeDtypeStruct((NCORE * NSUB * 16,), jnp.float32),
    mesh=plsc.VectorSubcoreMesh(core_axis_name="c", subcore_axis_name="s"),
    compiler_params=pltpu.CompilerParams(needs_layout_passes=False),
    scratch_types=[
        pltpu.VMEM((CHUNK,), jnp.float32),
        pltpu.VMEM((CHUNK,), jnp.float32),
        pltpu.VMEM((CHUNK,), jnp.int32),
        pltpu.VMEM((CHUNK,), jnp.float32),
        pltpu.VMEM((SUBB,), jnp.float32),
        pltpu.VMEM((SUBB,), jnp.float32),
        pltpu.VMEM((SUBB,), jnp.float32),
        pltpu.VMEM((SUBB,), jnp.float32),
        pltpu.VMEM((CHUNK,), jnp.float32),
        pltpu.VMEM((SLICE // 2,), jnp.float32),
        pltpu.VMEM((16,), jnp.float32),
        pltpu.VMEM((NSUB, 16), jnp.float32),
        pltpu.VMEM_SHARED((NB,), jnp.float32),
        pltpu.VMEM_SHARED((NB,), jnp.float32),
        pltpu.VMEM_SHARED((2 * NB,), jnp.float32),
        pltpu.VMEM_SHARED((NSUB, 16), jnp.float32),
    ],
)(_body)


def kernel(input, target):
    logits = input.reshape(NIMG, NSUB, CHUNK)
    labels = target.astype(jnp.float32).reshape(NIMG, NSUB, CHUNK)
    partials = _sc_call(logits, labels)
    return jnp.sum(partials) / NIMG
